# baseline (speedup 1.0000x reference)
import functools

import jax
import jax.numpy as jnp
from jax import lax
from jax.experimental import pallas as pl
from jax.experimental.pallas import tpu as pltpu
from jax.experimental.pallas import tpu_sc as plsc

_INFO = plsc.get_sparse_core_info()
_NC, _NS = _INFO.num_cores, _INFO.num_subcores
_NW = _NC * _NS

_B = 16384 * 50
_D = 128
_V = 83
_L = 16
_CHUNK = 128
_PER_W = _B // _NW
_ITERS = _PER_W // _CHUNK


def _make_lookup():
    mesh = plsc.VectorSubcoreMesh(core_axis_name="c", subcore_axis_name="s")

    @functools.partial(
        pl.kernel,
        mesh=mesh,
        out_type=jax.ShapeDtypeStruct((_B, _D), jnp.float32),
        scratch_types=[
            pltpu.VMEM((_V, _D), jnp.float32),
            pltpu.VMEM((_ITERS, _CHUNK), jnp.int32),
            pltpu.VMEM((_CHUNK, _D), jnp.float32),
            pltpu.VMEM((_CHUNK, _D), jnp.float32),
            pltpu.SemaphoreType.DMA,
        ],
    )
    def lookup(table_hbm, idx_hbm, out_hbm, table_v, idx_v, rows0, rows1, wsem):
        wid = lax.axis_index("s") * _NC + lax.axis_index("c")
        base = wid * _PER_W
        pltpu.sync_copy(table_hbm, table_v)
        pltpu.sync_copy(idx_hbm.at[pl.ds(wid * _ITERS, _ITERS)], idx_v)
        bufs = (rows0, rows1)

        def fill(c, buf):
            def grp(g, carry):
                v = idx_v[c, pl.ds(g * _L, _L)]
                for l in range(_L):
                    idx_r = v[l]
                    for d in range(_D // _L):
                        buf[g * _L + l, pl.ds(d * _L, _L)] = table_v[
                            idx_r, pl.ds(d * _L, _L)
                        ]
                return carry

            lax.fori_loop(0, _CHUNK // _L, grp, 0)

        def outer(c2, carry):
            for p in range(2):
                c = c2 * 2 + p
                buf = bufs[p]

                @pl.when(c >= 2)
                def _wait_buf_free():
                    pltpu.make_async_copy(
                        buf, out_hbm.at[pl.ds(base + (c - 2) * _CHUNK, _CHUNK)], wsem
                    ).wait()

                fill(c, buf)
                pltpu.async_copy(
                    buf, out_hbm.at[pl.ds(base + c * _CHUNK, _CHUNK)], wsem
                )
            return carry

        lax.fori_loop(0, _ITERS // 2, outer, 0)
        for p in range(2):
            pltpu.make_async_copy(
                bufs[p],
                out_hbm.at[pl.ds(base + (_ITERS - 2 + p) * _CHUNK, _CHUNK)],
                wsem,
            ).wait()

    return lookup


_lookup = _make_lookup()


def kernel(atom_number, embedding_list):
    idx = atom_number.reshape(_B // _CHUNK, _CHUNK)
    out = _lookup(embedding_list, idx)
    return out.reshape(atom_number.shape[0], atom_number.shape[1], _D)
